# Initial kernel scaffold; baseline (speedup 1.0000x reference)
import jax
import jax.numpy as jnp
from jax.experimental import pallas as pl


def kernel(X, y, W_rf, b_rf, pca_mean, pca_comp, W0, b0, W1, b1, W2, b2, W3, b3, nn_bias0, nn_bias1):
    raise NotImplementedError("write your pallas kernel here")


# speedup vs baseline: 10.1539x; 10.1539x over previous
import functools

import jax
import jax.numpy as jnp
from jax import lax
from jax.experimental import pallas as pl
from jax.experimental.pallas import tpu as pltpu
from jax.experimental.pallas import tpu_sc as plsc

_N = 4096
_D_IN = 128
_D_RF = 512
_D_PCA = 256
_HID = 256
_NCLS = 10
_CLIP = 5.0
_KBATCH = 128

_BR_MLP = 512
_BR_KNN = 256

_SC_NC = 2
_SC_NS = 16
_SC_NW = _SC_NC * _SC_NS
_SC_ROWS = _N // _SC_NW


def _mlp_body(xb, wrf, brf, mean, comp, w0, b0, w1, b1, w2, b2, w3, b3,
              logits, inter1):
    x = xb[...]
    h = lax.dot_general(x, wrf[...], (((1,), (1,)), ((), ())),
                        preferred_element_type=jnp.float32) + brf[...]
    h = h - mean[...]
    h = lax.dot_general(h, comp[...], (((1,), (1,)), ((), ())),
                        preferred_element_type=jnp.float32)
    h = jnp.clip(h, -_CLIP, _CLIP)
    res = h
    h = jnp.maximum(jnp.dot(h, w0[...], preferred_element_type=jnp.float32)
                    + b0[...], 0.0)
    h = jnp.dot(h, w1[...], preferred_element_type=jnp.float32) + b1[...]
    h = jnp.maximum(h + res, 0.0)
    h = jnp.maximum(jnp.dot(h, w2[...], preferred_element_type=jnp.float32)
                    + b2[...], 0.0)
    inter1[...] = h
    logits[...] = (jnp.dot(h, w3[...], preferred_element_type=jnp.float32)
                   + b3[...])


def _knn_body(ab, f, idx_ref, *, d):
    a = ab[...]
    ff = f[...]
    ones_k = jnp.ones((1, d), jnp.float32)
    g = lax.dot_general(a, ff, (((1,), (1,)), ((), ())),
                        preferred_element_type=jnp.float32)
    a2 = jnp.sum(a * a, axis=1, keepdims=True)
    b2 = lax.dot_general(ones_k, ff * ff, (((1,), (1,)), ((), ())),
                         preferred_element_type=jnp.float32)
    d2 = a2 + b2 - 2.0 * g
    d2 = jnp.maximum(d2, 0.0)
    col = lax.broadcasted_iota(jnp.int32, (_BR_KNN, _N), 1)
    rowm = lax.broadcasted_iota(jnp.int32, (_BR_KNN, _N), 0) % _KBATCH
    d2 = jnp.where(col == rowm, jnp.inf, d2)
    mv = jnp.min(d2, axis=1, keepdims=True)
    cand = jnp.where(d2 == mv, col, _N)
    idx = jnp.min(cand, axis=1, keepdims=True)
    idx_ref[...] = idx[None]


def _run_mlp(x, w_rf, b_rf, pca_mean, pca_comp, w0, b0, w1, b1, w2, b2,
             w3, b3):
    ni = _N // _BR_MLP
    full = lambda shape: pl.BlockSpec(shape, lambda i: (0,) * len(shape))
    return pl.pallas_call(
        _mlp_body,
        grid=(ni,),
        in_specs=[
            pl.BlockSpec((_BR_MLP, _D_IN), lambda i: (i, 0)),
            full((_D_RF, _D_IN)),
            full((1, _D_RF)),
            full((1, _D_RF)),
            full((_D_PCA, _D_RF)),
            full((_HID, _HID)),
            full((1, _HID)),
            full((_HID, _HID)),
            full((1, _HID)),
            full((_HID, _HID)),
            full((1, _HID)),
            full((_HID, _NCLS)),
            full((1, _NCLS)),
        ],
        out_specs=[
            pl.BlockSpec((_BR_MLP, _NCLS), lambda i: (i, 0)),
            pl.BlockSpec((_BR_MLP, _HID), lambda i: (i, 0)),
        ],
        out_shape=[
            jax.ShapeDtypeStruct((_N, _NCLS), jnp.float32),
            jax.ShapeDtypeStruct((_N, _HID), jnp.float32),
        ],
    )(x, w_rf, b_rf.reshape(1, _D_RF), pca_mean.reshape(1, _D_RF), pca_comp,
      w0, b0.reshape(1, _HID), w1, b1.reshape(1, _HID), w2,
      b2.reshape(1, _HID), w3, b3.reshape(1, _NCLS))


def _run_knn(f):
    d = f.shape[1]
    ni = _N // _BR_KNN
    idx = pl.pallas_call(
        functools.partial(_knn_body, d=d),
        grid=(ni,),
        in_specs=[
            pl.BlockSpec((_BR_KNN, d), lambda i: (i, 0)),
            pl.BlockSpec((_N, d), lambda i: (0, 0)),
        ],
        out_specs=pl.BlockSpec((1, _BR_KNN, 1), lambda i: (i, 0, 0)),
        out_shape=jax.ShapeDtypeStruct((ni, _BR_KNN, 1), jnp.int32),
    )(f, f)
    return idx.reshape(_N)


def _sc_bias_body(logits_hbm, idx_hbm, y_hbm, bias_hbm, out_hbm,
                  y_v, idx_v, out_v, bias_v):
    c = lax.axis_index("c")
    s = lax.axis_index("s")
    wid = s * _SC_NC + c
    base = wid * _SC_ROWS
    pltpu.sync_copy(y_hbm, y_v)
    pltpu.sync_copy(idx_hbm.at[pl.ds(base, _SC_ROWS)], idx_v)
    pltpu.sync_copy(logits_hbm.at[pl.ds(base * _NCLS, _SC_ROWS * _NCLS)],
                    out_v)
    pltpu.sync_copy(bias_hbm, bias_v)
    bias = bias_v[...]
    lane = lax.broadcasted_iota(jnp.int32, (16,), 0)
    for j in range(_SC_ROWS // 16):
        iv = idx_v[pl.ds(j * 16, 16)]
        p = plsc.load_gather(y_v, [iv])
        flat = (j * 16 + lane) * _NCLS + p
        plsc.addupdate_scatter(out_v, [flat], bias)
    pltpu.sync_copy(out_v, out_hbm.at[pl.ds(base * _NCLS, _SC_ROWS * _NCLS)])


@functools.cache
def _get_sc_bias_add():
    return pl.kernel(
        _sc_bias_body,
        mesh=plsc.VectorSubcoreMesh(core_axis_name="c",
                                    subcore_axis_name="s"),
        out_type=jax.ShapeDtypeStruct((_N * _NCLS,), jnp.float32),
        compiler_params=pltpu.CompilerParams(needs_layout_passes=False),
        scratch_types=[
            pltpu.VMEM((_N,), jnp.int32),
            pltpu.VMEM((_SC_ROWS,), jnp.int32),
            pltpu.VMEM((_SC_ROWS * _NCLS,), jnp.float32),
            pltpu.VMEM((16,), jnp.float32),
        ],
    )


def kernel(X, y, W_rf, b_rf, pca_mean, pca_comp, W0, b0, W1, b1, W2, b2,
           W3, b3, nn_bias0, nn_bias1):
    logits, inter1 = _run_mlp(X, W_rf, b_rf, pca_mean, pca_comp,
                              W0, b0, W1, b1, W2, b2, W3, b3)
    idx0 = _run_knn(X)
    idx1 = _run_knn(inter1)
    sc_bias_add = _get_sc_bias_add()
    out = sc_bias_add(logits.reshape(-1), idx0, y,
                      jnp.full((16,), nn_bias0, jnp.float32))
    out = sc_bias_add(out, idx1, y,
                      jnp.full((16,), nn_bias1, jnp.float32))
    return out.reshape(_N, _NCLS)


# speedup vs baseline: 11.0476x; 1.0880x over previous
import functools

import jax
import jax.numpy as jnp
from jax import lax
from jax.experimental import pallas as pl
from jax.experimental.pallas import tpu as pltpu
from jax.experimental.pallas import tpu_sc as plsc

_N = 4096
_D_IN = 128
_D_RF = 512
_D_PCA = 256
_HID = 256
_NCLS = 10
_CLIP = 5.0
_KBATCH = 128

_BR_MLP = 512
_BR_KNN = 256

_SC_NC = 2
_SC_NS = 16
_SC_NW = _SC_NC * _SC_NS
_SC_ROWS = _N // _SC_NW


def _mlp_body(xb, wrf, brf, mean, comp, w0, b0, w1, b1, w2, b2, w3, b3,
              logits, inter1):
    x = xb[...]
    h = lax.dot_general(x, wrf[...], (((1,), (1,)), ((), ())),
                        preferred_element_type=jnp.float32) + brf[...]
    h = h - mean[...]
    h = lax.dot_general(h, comp[...], (((1,), (1,)), ((), ())),
                        preferred_element_type=jnp.float32)
    h = jnp.clip(h, -_CLIP, _CLIP)
    res = h
    h = jnp.maximum(jnp.dot(h, w0[...], preferred_element_type=jnp.float32)
                    + b0[...], 0.0)
    h = jnp.dot(h, w1[...], preferred_element_type=jnp.float32) + b1[...]
    h = jnp.maximum(h + res, 0.0)
    h = jnp.maximum(jnp.dot(h, w2[...], preferred_element_type=jnp.float32)
                    + b2[...], 0.0)
    inter1[...] = h
    logits[...] = (jnp.dot(h, w3[...], preferred_element_type=jnp.float32)
                   + b3[...])


def _knn_body(ab, f, idx_ref, d2_ref, b2_ref, *, d):
    a = ab[...]
    ff = f[...]

    @pl.when(pl.program_id(0) == 0)
    def _():
        ones_k = jnp.ones((1, d), jnp.float32)
        b2_ref[...] = lax.dot_general(
            ones_k, ff * ff, (((1,), (1,)), ((), ())),
            preferred_element_type=jnp.float32)

    g = lax.dot_general(a, ff, (((1,), (1,)), ((), ())),
                        preferred_element_type=jnp.float32)
    a2 = jnp.sum(a * a, axis=1, keepdims=True)
    d2_ref[...] = jnp.maximum(a2 + b2_ref[...] - 2.0 * g, 0.0)

    colm = lax.broadcasted_iota(jnp.int32, (_BR_KNN, _KBATCH), 1)
    rowm = lax.broadcasted_iota(jnp.int32, (_BR_KNN, _KBATCH), 0) % _KBATCH
    d2_ref[:, :_KBATCH] = jnp.where(colm == rowm, jnp.inf,
                                    d2_ref[:, :_KBATCH])

    d2 = d2_ref[...]
    mv = jnp.min(d2, axis=1, keepdims=True)
    col = lax.broadcasted_iota(jnp.int32, (_BR_KNN, _N), 1)
    cand = jnp.where(d2 == mv, col, _N)
    idx = jnp.min(cand, axis=1, keepdims=True)
    idx_ref[...] = idx[None]


def _run_mlp(x, w_rf, b_rf, pca_mean, pca_comp, w0, b0, w1, b1, w2, b2,
             w3, b3):
    ni = _N // _BR_MLP
    full = lambda shape: pl.BlockSpec(shape, lambda i: (0,) * len(shape))
    return pl.pallas_call(
        _mlp_body,
        grid=(ni,),
        in_specs=[
            pl.BlockSpec((_BR_MLP, _D_IN), lambda i: (i, 0)),
            full((_D_RF, _D_IN)),
            full((1, _D_RF)),
            full((1, _D_RF)),
            full((_D_PCA, _D_RF)),
            full((_HID, _HID)),
            full((1, _HID)),
            full((_HID, _HID)),
            full((1, _HID)),
            full((_HID, _HID)),
            full((1, _HID)),
            full((_HID, _NCLS)),
            full((1, _NCLS)),
        ],
        out_specs=[
            pl.BlockSpec((_BR_MLP, _NCLS), lambda i: (i, 0)),
            pl.BlockSpec((_BR_MLP, _HID), lambda i: (i, 0)),
        ],
        out_shape=[
            jax.ShapeDtypeStruct((_N, _NCLS), jnp.float32),
            jax.ShapeDtypeStruct((_N, _HID), jnp.float32),
        ],
    )(x, w_rf, b_rf.reshape(1, _D_RF), pca_mean.reshape(1, _D_RF), pca_comp,
      w0, b0.reshape(1, _HID), w1, b1.reshape(1, _HID), w2,
      b2.reshape(1, _HID), w3, b3.reshape(1, _NCLS))


def _run_knn(f):
    d = f.shape[1]
    ni = _N // _BR_KNN
    idx = pl.pallas_call(
        functools.partial(_knn_body, d=d),
        grid=(ni,),
        in_specs=[
            pl.BlockSpec((_BR_KNN, d), lambda i: (i, 0)),
            pl.BlockSpec((_N, d), lambda i: (0, 0)),
        ],
        out_specs=pl.BlockSpec((1, _BR_KNN, 1), lambda i: (i, 0, 0)),
        out_shape=jax.ShapeDtypeStruct((ni, _BR_KNN, 1), jnp.int32),
        scratch_shapes=[
            pltpu.VMEM((_BR_KNN, _N), jnp.float32),
            pltpu.VMEM((1, _N), jnp.float32),
        ],
    )(f, f)
    return idx.reshape(_N)


def _sc_bias_body(logits_hbm, idx_hbm, y_hbm, bias_hbm, out_hbm,
                  y_v, idx_v, out_v, bias_v):
    c = lax.axis_index("c")
    s = lax.axis_index("s")
    wid = s * _SC_NC + c
    base = wid * _SC_ROWS
    pltpu.sync_copy(y_hbm, y_v)
    pltpu.sync_copy(idx_hbm.at[pl.ds(base, _SC_ROWS)], idx_v)
    pltpu.sync_copy(logits_hbm.at[pl.ds(base * _NCLS, _SC_ROWS * _NCLS)],
                    out_v)
    pltpu.sync_copy(bias_hbm, bias_v)
    bias = bias_v[...]
    lane = lax.broadcasted_iota(jnp.int32, (16,), 0)
    for j in range(_SC_ROWS // 16):
        iv = idx_v[pl.ds(j * 16, 16)]
        p = plsc.load_gather(y_v, [iv])
        flat = (j * 16 + lane) * _NCLS + p
        plsc.addupdate_scatter(out_v, [flat], bias)
    pltpu.sync_copy(out_v, out_hbm.at[pl.ds(base * _NCLS, _SC_ROWS * _NCLS)])


@functools.cache
def _get_sc_bias_add():
    return pl.kernel(
        _sc_bias_body,
        mesh=plsc.VectorSubcoreMesh(core_axis_name="c",
                                    subcore_axis_name="s"),
        out_type=jax.ShapeDtypeStruct((_N * _NCLS,), jnp.float32),
        compiler_params=pltpu.CompilerParams(needs_layout_passes=False),
        scratch_types=[
            pltpu.VMEM((_N,), jnp.int32),
            pltpu.VMEM((_SC_ROWS,), jnp.int32),
            pltpu.VMEM((_SC_ROWS * _NCLS,), jnp.float32),
            pltpu.VMEM((16,), jnp.float32),
        ],
    )


def kernel(X, y, W_rf, b_rf, pca_mean, pca_comp, W0, b0, W1, b1, W2, b2,
           W3, b3, nn_bias0, nn_bias1):
    logits, inter1 = _run_mlp(X, W_rf, b_rf, pca_mean, pca_comp,
                              W0, b0, W1, b1, W2, b2, W3, b3)
    idx0 = _run_knn(X)
    idx1 = _run_knn(inter1)
    sc_bias_add = _get_sc_bias_add()
    out = sc_bias_add(logits.reshape(-1), idx0, y,
                      jnp.full((16,), nn_bias0, jnp.float32))
    out = sc_bias_add(out, idx1, y,
                      jnp.full((16,), nn_bias1, jnp.float32))
    return out.reshape(_N, _NCLS)


# speedup vs baseline: 11.8735x; 1.0748x over previous
import functools

import jax
import jax.numpy as jnp
from jax import lax
from jax.experimental import pallas as pl
from jax.experimental.pallas import tpu as pltpu
from jax.experimental.pallas import tpu_sc as plsc

_N = 4096
_D_IN = 128
_D_RF = 512
_D_PCA = 256
_HID = 256
_NCLS = 10
_CLIP = 5.0
_KBATCH = 128

_BR_MLP = 512
_BR_KNN = 512

_SC_NC = 2
_SC_NS = 16
_SC_NW = _SC_NC * _SC_NS
_SC_ROWS = _N // _SC_NW


def _mlp_body(xb, wrf, brf, mean, comp, w0, b0, w1, b1, w2, b2, w3, b3,
              logits, inter1):
    x = xb[...]
    h = lax.dot_general(x, wrf[...], (((1,), (1,)), ((), ())),
                        preferred_element_type=jnp.float32) + brf[...]
    h = h - mean[...]
    h = lax.dot_general(h, comp[...], (((1,), (1,)), ((), ())),
                        preferred_element_type=jnp.float32)
    h = jnp.clip(h, -_CLIP, _CLIP)
    res = h
    h = jnp.maximum(jnp.dot(h, w0[...], preferred_element_type=jnp.float32)
                    + b0[...], 0.0)
    h = jnp.dot(h, w1[...], preferred_element_type=jnp.float32) + b1[...]
    h = jnp.maximum(h + res, 0.0)
    h = jnp.maximum(jnp.dot(h, w2[...], preferred_element_type=jnp.float32)
                    + b2[...], 0.0)
    inter1[...] = h
    logits[...] = (jnp.dot(h, w3[...], preferred_element_type=jnp.float32)
                   + b3[...])


def _knn_body(ab, f, idx_ref, d2_ref, b2_ref, *, d):
    a = ab[...]
    ff = f[...]

    @pl.when(pl.program_id(0) == 0)
    def _():
        ones_k = jnp.ones((1, d), jnp.float32)
        b2_ref[...] = lax.dot_general(
            ones_k, ff * ff, (((1,), (1,)), ((), ())),
            preferred_element_type=jnp.float32)

    g = lax.dot_general(a, ff, (((1,), (1,)), ((), ())),
                        preferred_element_type=jnp.float32)
    a2 = jnp.sum(a * a, axis=1, keepdims=True)
    d2_ref[...] = jnp.maximum(a2 + b2_ref[...] - 2.0 * g, 0.0)

    colm = lax.broadcasted_iota(jnp.int32, (_BR_KNN, _KBATCH), 1)
    rowm = lax.broadcasted_iota(jnp.int32, (_BR_KNN, _KBATCH), 0) % _KBATCH
    d2_ref[:, :_KBATCH] = jnp.where(colm == rowm, jnp.inf,
                                    d2_ref[:, :_KBATCH])

    d2 = d2_ref[...]
    mv = jnp.min(d2, axis=1, keepdims=True)
    col = lax.broadcasted_iota(jnp.int32, (_BR_KNN, _N), 1)
    cand = jnp.where(d2 == mv, col, _N)
    idx = jnp.min(cand, axis=1, keepdims=True)
    idx_ref[...] = idx[None]


def _run_mlp(x, w_rf, b_rf, pca_mean, pca_comp, w0, b0, w1, b1, w2, b2,
             w3, b3):
    ni = _N // _BR_MLP
    full = lambda shape: pl.BlockSpec(shape, lambda i: (0,) * len(shape))
    return pl.pallas_call(
        _mlp_body,
        grid=(ni,),
        in_specs=[
            pl.BlockSpec((_BR_MLP, _D_IN), lambda i: (i, 0)),
            full((_D_RF, _D_IN)),
            full((1, _D_RF)),
            full((1, _D_RF)),
            full((_D_PCA, _D_RF)),
            full((_HID, _HID)),
            full((1, _HID)),
            full((_HID, _HID)),
            full((1, _HID)),
            full((_HID, _HID)),
            full((1, _HID)),
            full((_HID, _NCLS)),
            full((1, _NCLS)),
        ],
        out_specs=[
            pl.BlockSpec((_BR_MLP, _NCLS), lambda i: (i, 0)),
            pl.BlockSpec((_BR_MLP, _HID), lambda i: (i, 0)),
        ],
        out_shape=[
            jax.ShapeDtypeStruct((_N, _NCLS), jnp.float32),
            jax.ShapeDtypeStruct((_N, _HID), jnp.float32),
        ],
    )(x, w_rf, b_rf.reshape(1, _D_RF), pca_mean.reshape(1, _D_RF), pca_comp,
      w0, b0.reshape(1, _HID), w1, b1.reshape(1, _HID), w2,
      b2.reshape(1, _HID), w3, b3.reshape(1, _NCLS))


def _run_knn(f):
    d = f.shape[1]
    ni = _N // _BR_KNN
    idx = pl.pallas_call(
        functools.partial(_knn_body, d=d),
        grid=(ni,),
        in_specs=[
            pl.BlockSpec((_BR_KNN, d), lambda i: (i, 0)),
            pl.BlockSpec((_N, d), lambda i: (0, 0)),
        ],
        out_specs=pl.BlockSpec((1, _BR_KNN, 1), lambda i: (i, 0, 0)),
        out_shape=jax.ShapeDtypeStruct((ni, _BR_KNN, 1), jnp.int32),
        scratch_shapes=[
            pltpu.VMEM((_BR_KNN, _N), jnp.float32),
            pltpu.VMEM((1, _N), jnp.float32),
        ],
    )(f, f)
    return idx.reshape(_N)


def _sc_bias_body(logits_hbm, idx_hbm, y_hbm, bias_hbm, out_hbm,
                  y_v, idx_v, out_v, bias_v):
    c = lax.axis_index("c")
    s = lax.axis_index("s")
    wid = s * _SC_NC + c
    base = wid * _SC_ROWS
    pltpu.sync_copy(y_hbm, y_v)
    pltpu.sync_copy(idx_hbm.at[pl.ds(base, _SC_ROWS)], idx_v)
    pltpu.sync_copy(logits_hbm.at[pl.ds(base * _NCLS, _SC_ROWS * _NCLS)],
                    out_v)
    pltpu.sync_copy(bias_hbm, bias_v)
    bias = bias_v[...]
    lane = lax.broadcasted_iota(jnp.int32, (16,), 0)
    for j in range(_SC_ROWS // 16):
        iv = idx_v[pl.ds(j * 16, 16)]
        p = plsc.load_gather(y_v, [iv])
        flat = (j * 16 + lane) * _NCLS + p
        plsc.addupdate_scatter(out_v, [flat], bias)
    pltpu.sync_copy(out_v, out_hbm.at[pl.ds(base * _NCLS, _SC_ROWS * _NCLS)])


@functools.cache
def _get_sc_bias_add():
    return pl.kernel(
        _sc_bias_body,
        mesh=plsc.VectorSubcoreMesh(core_axis_name="c",
                                    subcore_axis_name="s"),
        out_type=jax.ShapeDtypeStruct((_N * _NCLS,), jnp.float32),
        compiler_params=pltpu.CompilerParams(needs_layout_passes=False),
        scratch_types=[
            pltpu.VMEM((_N,), jnp.int32),
            pltpu.VMEM((_SC_ROWS,), jnp.int32),
            pltpu.VMEM((_SC_ROWS * _NCLS,), jnp.float32),
            pltpu.VMEM((16,), jnp.float32),
        ],
    )


def kernel(X, y, W_rf, b_rf, pca_mean, pca_comp, W0, b0, W1, b1, W2, b2,
           W3, b3, nn_bias0, nn_bias1):
    logits, inter1 = _run_mlp(X, W_rf, b_rf, pca_mean, pca_comp,
                              W0, b0, W1, b1, W2, b2, W3, b3)
    idx0 = _run_knn(X)
    idx1 = _run_knn(inter1)
    sc_bias_add = _get_sc_bias_add()
    out = sc_bias_add(logits.reshape(-1), idx0, y,
                      jnp.full((16,), nn_bias0, jnp.float32))
    out = sc_bias_add(out, idx1, y,
                      jnp.full((16,), nn_bias1, jnp.float32))
    return out.reshape(_N, _NCLS)


# speedup vs baseline: 12.7705x; 1.0755x over previous
import functools

import jax
import jax.numpy as jnp
from jax import lax
from jax.experimental import pallas as pl
from jax.experimental.pallas import tpu as pltpu
from jax.experimental.pallas import tpu_sc as plsc

_N = 4096
_D_IN = 128
_D_RF = 512
_D_PCA = 256
_HID = 256
_NCLS = 10
_CLIP = 5.0
_KBATCH = 128

_BR_MLP = 512
_BR_KNN = 512

_SC_NC = 2
_SC_NS = 16
_SC_NW = _SC_NC * _SC_NS
_SC_ROWS = _N // _SC_NW


def _mlp_body(xb, wrf, brf, mean, comp, w0, b0, w1, b1, w2, b2, w3, b3,
              logits, inter1):
    x = xb[...]
    h = lax.dot_general(x, wrf[...], (((1,), (1,)), ((), ())),
                        preferred_element_type=jnp.float32) + brf[...]
    h = h - mean[...]
    h = lax.dot_general(h, comp[...], (((1,), (1,)), ((), ())),
                        preferred_element_type=jnp.float32)
    h = jnp.clip(h, -_CLIP, _CLIP)
    res = h
    h = jnp.maximum(jnp.dot(h, w0[...], preferred_element_type=jnp.float32)
                    + b0[...], 0.0)
    h = jnp.dot(h, w1[...], preferred_element_type=jnp.float32) + b1[...]
    h = jnp.maximum(h + res, 0.0)
    h = jnp.maximum(jnp.dot(h, w2[...], preferred_element_type=jnp.float32)
                    + b2[...], 0.0)
    inter1[...] = h
    logits[...] = (jnp.dot(h, w3[...], preferred_element_type=jnp.float32)
                   + b3[...])


def _knn_body(ab, f, idx_ref, d2_ref, b2_ref, *, d):
    a = ab[...]
    ff = f[...]

    @pl.when(pl.program_id(0) == 0)
    def _():
        ones_k = jnp.ones((1, d), jnp.float32)
        b2_ref[...] = lax.dot_general(
            ones_k, ff * ff, (((1,), (1,)), ((), ())),
            preferred_element_type=jnp.float32)

    g = lax.dot_general(a, ff, (((1,), (1,)), ((), ())),
                        preferred_element_type=jnp.float32)
    a2 = jnp.sum(a * a, axis=1, keepdims=True)
    d2_ref[...] = a2 + b2_ref[...] - 2.0 * g

    colm = lax.broadcasted_iota(jnp.int32, (_BR_KNN, _KBATCH), 1)
    rowm = lax.broadcasted_iota(jnp.int32, (_BR_KNN, _KBATCH), 0) % _KBATCH
    d2_ref[:, :_KBATCH] = jnp.where(colm == rowm, jnp.inf,
                                    d2_ref[:, :_KBATCH])

    d2 = d2_ref[...]
    mv = jnp.min(d2, axis=1, keepdims=True)
    col = lax.broadcasted_iota(jnp.int32, (_BR_KNN, _N), 1)
    cand = jnp.where(d2 == mv, col, _N)
    idx = jnp.min(cand, axis=1, keepdims=True)
    idx_ref[...] = idx[None]


def _run_mlp(x, w_rf, b_rf, pca_mean, pca_comp, w0, b0, w1, b1, w2, b2,
             w3, b3):
    ni = _N // _BR_MLP
    full = lambda shape: pl.BlockSpec(shape, lambda i: (0,) * len(shape))
    return pl.pallas_call(
        _mlp_body,
        grid=(ni,),
        in_specs=[
            pl.BlockSpec((_BR_MLP, _D_IN), lambda i: (i, 0)),
            full((_D_RF, _D_IN)),
            full((1, _D_RF)),
            full((1, _D_RF)),
            full((_D_PCA, _D_RF)),
            full((_HID, _HID)),
            full((1, _HID)),
            full((_HID, _HID)),
            full((1, _HID)),
            full((_HID, _HID)),
            full((1, _HID)),
            full((_HID, _NCLS)),
            full((1, _NCLS)),
        ],
        out_specs=[
            pl.BlockSpec((_BR_MLP, _NCLS), lambda i: (i, 0)),
            pl.BlockSpec((_BR_MLP, _HID), lambda i: (i, 0)),
        ],
        out_shape=[
            jax.ShapeDtypeStruct((_N, _NCLS), jnp.float32),
            jax.ShapeDtypeStruct((_N, _HID), jnp.float32),
        ],
    )(x, w_rf, b_rf.reshape(1, _D_RF), pca_mean.reshape(1, _D_RF), pca_comp,
      w0, b0.reshape(1, _HID), w1, b1.reshape(1, _HID), w2,
      b2.reshape(1, _HID), w3, b3.reshape(1, _NCLS))


def _run_knn(f):
    d = f.shape[1]
    ni = _N // _BR_KNN
    idx = pl.pallas_call(
        functools.partial(_knn_body, d=d),
        grid=(ni,),
        in_specs=[
            pl.BlockSpec((_BR_KNN, d), lambda i: (i, 0)),
            pl.BlockSpec((_N, d), lambda i: (0, 0)),
        ],
        out_specs=pl.BlockSpec((1, _BR_KNN, 1), lambda i: (i, 0, 0)),
        out_shape=jax.ShapeDtypeStruct((ni, _BR_KNN, 1), jnp.int32),
        scratch_shapes=[
            pltpu.VMEM((_BR_KNN, _N), jnp.float32),
            pltpu.VMEM((1, _N), jnp.float32),
        ],
    )(f, f)
    return idx.reshape(_N)


def _sc_bias_body(logits_hbm, idx0_hbm, idx1_hbm, y_hbm, bias_hbm, out_hbm,
                  y_v, idx_v, out_v, bias_v):
    c = lax.axis_index("c")
    s = lax.axis_index("s")
    wid = s * _SC_NC + c
    base = wid * _SC_ROWS
    pltpu.sync_copy(y_hbm, y_v)
    pltpu.sync_copy(logits_hbm.at[pl.ds(base * _NCLS, _SC_ROWS * _NCLS)],
                    out_v)
    pltpu.sync_copy(bias_hbm, bias_v)
    lane = lax.broadcasted_iota(jnp.int32, (16,), 0)
    for which, idx_hbm in enumerate((idx0_hbm, idx1_hbm)):
        pltpu.sync_copy(idx_hbm.at[pl.ds(base, _SC_ROWS)], idx_v)
        bias = bias_v[pl.ds(which * 16, 16)]
        for j in range(_SC_ROWS // 16):
            iv = idx_v[pl.ds(j * 16, 16)]
            p = plsc.load_gather(y_v, [iv])
            flat = (j * 16 + lane) * _NCLS + p
            plsc.addupdate_scatter(out_v, [flat], bias)
    pltpu.sync_copy(out_v, out_hbm.at[pl.ds(base * _NCLS, _SC_ROWS * _NCLS)])


@functools.cache
def _get_sc_bias_add():
    return pl.kernel(
        _sc_bias_body,
        mesh=plsc.VectorSubcoreMesh(core_axis_name="c",
                                    subcore_axis_name="s"),
        out_type=jax.ShapeDtypeStruct((_N * _NCLS,), jnp.float32),
        compiler_params=pltpu.CompilerParams(needs_layout_passes=False),
        scratch_types=[
            pltpu.VMEM((_N,), jnp.int32),
            pltpu.VMEM((_SC_ROWS,), jnp.int32),
            pltpu.VMEM((_SC_ROWS * _NCLS,), jnp.float32),
            pltpu.VMEM((32,), jnp.float32),
        ],
    )


def kernel(X, y, W_rf, b_rf, pca_mean, pca_comp, W0, b0, W1, b1, W2, b2,
           W3, b3, nn_bias0, nn_bias1):
    logits, inter1 = _run_mlp(X, W_rf, b_rf, pca_mean, pca_comp,
                              W0, b0, W1, b1, W2, b2, W3, b3)
    idx0 = _run_knn(X)
    idx1 = _run_knn(inter1)
    sc_bias_add = _get_sc_bias_add()
    biases = jnp.concatenate([jnp.full((16,), nn_bias0, jnp.float32),
                              jnp.full((16,), nn_bias1, jnp.float32)])
    out = sc_bias_add(logits.reshape(-1), idx0, idx1, y, biases)
    return out.reshape(_N, _NCLS)


# speedup vs baseline: 13.2899x; 1.0407x over previous
import functools

import jax
import jax.numpy as jnp
from jax import lax
from jax.experimental import pallas as pl
from jax.experimental.pallas import tpu as pltpu
from jax.experimental.pallas import tpu_sc as plsc

_N = 4096
_D_IN = 128
_D_RF = 512
_D_PCA = 256
_HID = 256
_NCLS = 10
_CLIP = 5.0
_KBATCH = 128

_BR = 512
_NB = _N // _BR

_SC_NC = 2
_SC_NS = 16
_SC_NW = _SC_NC * _SC_NS
_SC_ROWS = _N // _SC_NW


def _knn_block(a, ff, idx_ref, row0, d2_ref, b2_ref):
    g = lax.dot_general(a, ff, (((1,), (1,)), ((), ())),
                        preferred_element_type=jnp.float32)
    a2 = jnp.sum(a * a, axis=1, keepdims=True)
    d2_ref[...] = a2 + b2_ref[...] - 2.0 * g

    colm = lax.broadcasted_iota(jnp.int32, (_BR, _KBATCH), 1)
    rowm = lax.broadcasted_iota(jnp.int32, (_BR, _KBATCH), 0) % _KBATCH
    d2_ref[:, :_KBATCH] = jnp.where(colm == rowm, jnp.inf,
                                    d2_ref[:, :_KBATCH])

    d2 = d2_ref[...]
    mv = jnp.min(d2, axis=1, keepdims=True)
    col = lax.broadcasted_iota(jnp.int32, (_BR, _N), 1)
    cand = jnp.where(d2 == mv, col, _N)
    idx = jnp.min(cand, axis=1, keepdims=True)
    idx_ref[pl.ds(row0, _BR), :] = idx


def _norms_row(ff):
    ones_k = jnp.ones((1, ff.shape[1]), jnp.float32)
    return lax.dot_general(ones_k, ff * ff, (((1,), (1,)), ((), ())),
                           preferred_element_type=jnp.float32)


def _fused_body(xb, xf, wrf, brf, mean, comp, w0, b0, w1, b1, w2, b2, w3, b3,
                logits_ref, idx0_ref, idx1_ref, inter1_ref, d2_ref, b2_ref):
    i = pl.program_id(0)

    @pl.when(i < _NB)
    def _phase_a():
        x = xb[...]
        h = lax.dot_general(x, wrf[...], (((1,), (1,)), ((), ())),
                            preferred_element_type=jnp.float32) + brf[...]
        h = h - mean[...]
        h = lax.dot_general(h, comp[...], (((1,), (1,)), ((), ())),
                            preferred_element_type=jnp.float32)
        h = jnp.clip(h, -_CLIP, _CLIP)
        res = h
        h = jnp.maximum(
            jnp.dot(h, w0[...], preferred_element_type=jnp.float32)
            + b0[...], 0.0)
        h = jnp.dot(h, w1[...], preferred_element_type=jnp.float32) + b1[...]
        h = jnp.maximum(h + res, 0.0)
        h = jnp.maximum(
            jnp.dot(h, w2[...], preferred_element_type=jnp.float32)
            + b2[...], 0.0)
        inter1_ref[pl.ds(i * _BR, _BR), :] = h
        logits_ref[pl.ds(i * _BR, _BR), :] = (
            jnp.dot(h, w3[...], preferred_element_type=jnp.float32)
            + b3[...])

        @pl.when(i == 0)
        def _():
            b2_ref[...] = _norms_row(xf[...])

        _knn_block(x, xf[...], idx0_ref, i * _BR, d2_ref, b2_ref)

    @pl.when(i >= _NB)
    def _phase_b():
        j = i - _NB
        hh = inter1_ref[...]

        @pl.when(i == _NB)
        def _():
            b2_ref[...] = _norms_row(hh)

        a = inter1_ref[pl.ds(j * _BR, _BR), :]
        _knn_block(a, hh, idx1_ref, j * _BR, d2_ref, b2_ref)


def _run_fused(x, w_rf, b_rf, pca_mean, pca_comp, w0, b0, w1, b1, w2, b2,
               w3, b3):
    full = lambda shape: pl.BlockSpec(shape, lambda i: (0,) * len(shape))
    blk = lambda i: (i % _NB, 0)
    logits, idx0, idx1 = pl.pallas_call(
        _fused_body,
        grid=(2 * _NB,),
        in_specs=[
            pl.BlockSpec((_BR, _D_IN), blk),
            pl.BlockSpec((_N, _D_IN), lambda i: (0, 0)),
            full((_D_RF, _D_IN)),
            full((1, _D_RF)),
            full((1, _D_RF)),
            full((_D_PCA, _D_RF)),
            full((_HID, _HID)),
            full((1, _HID)),
            full((_HID, _HID)),
            full((1, _HID)),
            full((_HID, _HID)),
            full((1, _HID)),
            full((_HID, _NCLS)),
            full((1, _NCLS)),
        ],
        out_specs=[
            pl.BlockSpec((_N, _NCLS), lambda i: (0, 0)),
            pl.BlockSpec((_N, 1), lambda i: (0, 0)),
            pl.BlockSpec((_N, 1), lambda i: (0, 0)),
        ],
        out_shape=[
            jax.ShapeDtypeStruct((_N, _NCLS), jnp.float32),
            jax.ShapeDtypeStruct((_N, 1), jnp.int32),
            jax.ShapeDtypeStruct((_N, 1), jnp.int32),
        ],
        scratch_shapes=[
            pltpu.VMEM((_N, _HID), jnp.float32),
            pltpu.VMEM((_BR, _N), jnp.float32),
            pltpu.VMEM((1, _N), jnp.float32),
        ],
    )(x, x, w_rf, b_rf.reshape(1, _D_RF), pca_mean.reshape(1, _D_RF),
      pca_comp, w0, b0.reshape(1, _HID), w1, b1.reshape(1, _HID), w2,
      b2.reshape(1, _HID), w3, b3.reshape(1, _NCLS))
    return logits, idx0.reshape(_N), idx1.reshape(_N)


def _sc_bias_body(logits_hbm, idx0_hbm, idx1_hbm, y_hbm, bias_hbm, out_hbm,
                  y_v, idx_v, out_v, bias_v):
    c = lax.axis_index("c")
    s = lax.axis_index("s")
    wid = s * _SC_NC + c
    base = wid * _SC_ROWS
    pltpu.sync_copy(y_hbm, y_v)
    pltpu.sync_copy(logits_hbm.at[pl.ds(base * _NCLS, _SC_ROWS * _NCLS)],
                    out_v)
    pltpu.sync_copy(bias_hbm, bias_v)
    lane = lax.broadcasted_iota(jnp.int32, (16,), 0)
    for which, idx_hbm in enumerate((idx0_hbm, idx1_hbm)):
        pltpu.sync_copy(idx_hbm.at[pl.ds(base, _SC_ROWS)], idx_v)
        bias = bias_v[pl.ds(which * 16, 16)]
        for j in range(_SC_ROWS // 16):
            iv = idx_v[pl.ds(j * 16, 16)]
            p = plsc.load_gather(y_v, [iv])
            flat = (j * 16 + lane) * _NCLS + p
            plsc.addupdate_scatter(out_v, [flat], bias)
    pltpu.sync_copy(out_v, out_hbm.at[pl.ds(base * _NCLS, _SC_ROWS * _NCLS)])


@functools.cache
def _get_sc_bias_add():
    return pl.kernel(
        _sc_bias_body,
        mesh=plsc.VectorSubcoreMesh(core_axis_name="c",
                                    subcore_axis_name="s"),
        out_type=jax.ShapeDtypeStruct((_N * _NCLS,), jnp.float32),
        compiler_params=pltpu.CompilerParams(needs_layout_passes=False),
        scratch_types=[
            pltpu.VMEM((_N,), jnp.int32),
            pltpu.VMEM((_SC_ROWS,), jnp.int32),
            pltpu.VMEM((_SC_ROWS * _NCLS,), jnp.float32),
            pltpu.VMEM((32,), jnp.float32),
        ],
    )


def kernel(X, y, W_rf, b_rf, pca_mean, pca_comp, W0, b0, W1, b1, W2, b2,
           W3, b3, nn_bias0, nn_bias1):
    logits, idx0, idx1 = _run_fused(X, W_rf, b_rf, pca_mean, pca_comp,
                                    W0, b0, W1, b1, W2, b2, W3, b3)
    sc_bias_add = _get_sc_bias_add()
    biases = jnp.concatenate([jnp.full((16,), nn_bias0, jnp.float32),
                              jnp.full((16,), nn_bias1, jnp.float32)])
    out = sc_bias_add(logits.reshape(-1), idx0, idx1, y, biases)
    return out.reshape(_N, _NCLS)


# speedup vs baseline: 14.3287x; 1.0782x over previous
import functools

import jax
import jax.numpy as jnp
from jax import lax
from jax.experimental import pallas as pl
from jax.experimental.pallas import tpu as pltpu
from jax.experimental.pallas import tpu_sc as plsc

_N = 4096
_D_IN = 128
_D_RF = 512
_D_PCA = 256
_HID = 256
_NCLS = 10
_CLIP = 5.0
_KBATCH = 128

_BR = 1024
_NB = _N // _BR

_SC_NC = 2
_SC_NS = 16
_SC_NW = _SC_NC * _SC_NS
_SC_ROWS = _N // _SC_NW


def _knn_block(a, ff, idx_ref, row0, d2_ref, b2_ref):
    g = lax.dot_general(a, ff, (((1,), (1,)), ((), ())),
                        preferred_element_type=jnp.float32)
    a2 = jnp.sum(a * a, axis=1, keepdims=True)
    d2_ref[...] = a2 + b2_ref[...] - 2.0 * g

    colm = lax.broadcasted_iota(jnp.int32, (_BR, _KBATCH), 1)
    rowm = lax.broadcasted_iota(jnp.int32, (_BR, _KBATCH), 0) % _KBATCH
    d2_ref[:, :_KBATCH] = jnp.where(colm == rowm, jnp.inf,
                                    d2_ref[:, :_KBATCH])

    d2 = d2_ref[...]
    mv = jnp.min(d2, axis=1, keepdims=True)
    col = lax.broadcasted_iota(jnp.int32, (_BR, _N), 1)
    cand = jnp.where(d2 == mv, col, _N)
    idx = jnp.min(cand, axis=1, keepdims=True)
    idx_ref[pl.ds(row0, _BR), :] = idx


def _norms_row(ff):
    ones_k = jnp.ones((1, ff.shape[1]), jnp.float32)
    return lax.dot_general(ones_k, ff * ff, (((1,), (1,)), ((), ())),
                           preferred_element_type=jnp.float32)


def _fused_body(xb, xf, wrf, brf, mean, comp, w0, b0, w1, b1, w2, b2, w3, b3,
                logits_ref, idx0_ref, idx1_ref, inter1_ref, d2_ref, b2_ref):
    i = pl.program_id(0)

    @pl.when(i < _NB)
    def _phase_a():
        x = xb[...]
        h = lax.dot_general(x, wrf[...], (((1,), (1,)), ((), ())),
                            preferred_element_type=jnp.float32) + brf[...]
        h = h - mean[...]
        h = lax.dot_general(h, comp[...], (((1,), (1,)), ((), ())),
                            preferred_element_type=jnp.float32)
        h = jnp.clip(h, -_CLIP, _CLIP)
        res = h
        h = jnp.maximum(
            jnp.dot(h, w0[...], preferred_element_type=jnp.float32)
            + b0[...], 0.0)
        h = jnp.dot(h, w1[...], preferred_element_type=jnp.float32) + b1[...]
        h = jnp.maximum(h + res, 0.0)
        h = jnp.maximum(
            jnp.dot(h, w2[...], preferred_element_type=jnp.float32)
            + b2[...], 0.0)
        inter1_ref[pl.ds(i * _BR, _BR), :] = h
        logits_ref[pl.ds(i * _BR, _BR), :] = (
            jnp.dot(h, w3[...], preferred_element_type=jnp.float32)
            + b3[...])

        @pl.when(i == 0)
        def _():
            b2_ref[...] = _norms_row(xf[...])

        _knn_block(x, xf[...], idx0_ref, i * _BR, d2_ref, b2_ref)

    @pl.when(i >= _NB)
    def _phase_b():
        j = i - _NB
        hh = inter1_ref[...]

        @pl.when(i == _NB)
        def _():
            b2_ref[...] = _norms_row(hh)

        a = inter1_ref[pl.ds(j * _BR, _BR), :]
        _knn_block(a, hh, idx1_ref, j * _BR, d2_ref, b2_ref)


def _run_fused(x, w_rf, b_rf, pca_mean, pca_comp, w0, b0, w1, b1, w2, b2,
               w3, b3):
    full = lambda shape: pl.BlockSpec(shape, lambda i: (0,) * len(shape))
    blk = lambda i: (i % _NB, 0)
    logits, idx0, idx1 = pl.pallas_call(
        _fused_body,
        grid=(2 * _NB,),
        in_specs=[
            pl.BlockSpec((_BR, _D_IN), blk),
            pl.BlockSpec((_N, _D_IN), lambda i: (0, 0)),
            full((_D_RF, _D_IN)),
            full((1, _D_RF)),
            full((1, _D_RF)),
            full((_D_PCA, _D_RF)),
            full((_HID, _HID)),
            full((1, _HID)),
            full((_HID, _HID)),
            full((1, _HID)),
            full((_HID, _HID)),
            full((1, _HID)),
            full((_HID, _NCLS)),
            full((1, _NCLS)),
        ],
        out_specs=[
            pl.BlockSpec((_N, _NCLS), lambda i: (0, 0)),
            pl.BlockSpec((_N, 1), lambda i: (0, 0)),
            pl.BlockSpec((_N, 1), lambda i: (0, 0)),
        ],
        out_shape=[
            jax.ShapeDtypeStruct((_N, _NCLS), jnp.float32),
            jax.ShapeDtypeStruct((_N, 1), jnp.int32),
            jax.ShapeDtypeStruct((_N, 1), jnp.int32),
        ],
        scratch_shapes=[
            pltpu.VMEM((_N, _HID), jnp.float32),
            pltpu.VMEM((_BR, _N), jnp.float32),
            pltpu.VMEM((1, _N), jnp.float32),
        ],
    )(x, x, w_rf, b_rf.reshape(1, _D_RF), pca_mean.reshape(1, _D_RF),
      pca_comp, w0, b0.reshape(1, _HID), w1, b1.reshape(1, _HID), w2,
      b2.reshape(1, _HID), w3, b3.reshape(1, _NCLS))
    return logits, idx0.reshape(_N), idx1.reshape(_N)


def _sc_bias_body(logits_hbm, idx0_hbm, idx1_hbm, y_hbm, bias_hbm, out_hbm,
                  y_v, idx_v, out_v, bias_v):
    c = lax.axis_index("c")
    s = lax.axis_index("s")
    wid = s * _SC_NC + c
    base = wid * _SC_ROWS
    pltpu.sync_copy(y_hbm, y_v)
    pltpu.sync_copy(logits_hbm.at[pl.ds(base * _NCLS, _SC_ROWS * _NCLS)],
                    out_v)
    pltpu.sync_copy(bias_hbm, bias_v)
    lane = lax.broadcasted_iota(jnp.int32, (16,), 0)
    for which, idx_hbm in enumerate((idx0_hbm, idx1_hbm)):
        pltpu.sync_copy(idx_hbm.at[pl.ds(base, _SC_ROWS)], idx_v)
        bias = bias_v[pl.ds(which * 16, 16)]
        for j in range(_SC_ROWS // 16):
            iv = idx_v[pl.ds(j * 16, 16)]
            p = plsc.load_gather(y_v, [iv])
            flat = (j * 16 + lane) * _NCLS + p
            plsc.addupdate_scatter(out_v, [flat], bias)
    pltpu.sync_copy(out_v, out_hbm.at[pl.ds(base * _NCLS, _SC_ROWS * _NCLS)])


@functools.cache
def _get_sc_bias_add():
    return pl.kernel(
        _sc_bias_body,
        mesh=plsc.VectorSubcoreMesh(core_axis_name="c",
                                    subcore_axis_name="s"),
        out_type=jax.ShapeDtypeStruct((_N * _NCLS,), jnp.float32),
        compiler_params=pltpu.CompilerParams(needs_layout_passes=False),
        scratch_types=[
            pltpu.VMEM((_N,), jnp.int32),
            pltpu.VMEM((_SC_ROWS,), jnp.int32),
            pltpu.VMEM((_SC_ROWS * _NCLS,), jnp.float32),
            pltpu.VMEM((32,), jnp.float32),
        ],
    )


def kernel(X, y, W_rf, b_rf, pca_mean, pca_comp, W0, b0, W1, b1, W2, b2,
           W3, b3, nn_bias0, nn_bias1):
    logits, idx0, idx1 = _run_fused(X, W_rf, b_rf, pca_mean, pca_comp,
                                    W0, b0, W1, b1, W2, b2, W3, b3)
    sc_bias_add = _get_sc_bias_add()
    biases = jnp.concatenate([jnp.full((16,), nn_bias0, jnp.float32),
                              jnp.full((16,), nn_bias1, jnp.float32)])
    out = sc_bias_add(logits.reshape(-1), idx0, idx1, y, biases)
    return out.reshape(_N, _NCLS)


# speedup vs baseline: 14.7836x; 1.0317x over previous
import functools

import jax
import jax.numpy as jnp
from jax import lax
from jax.experimental import pallas as pl
from jax.experimental.pallas import tpu as pltpu
from jax.experimental.pallas import tpu_sc as plsc

_N = 4096
_D_IN = 128
_D_RF = 512
_D_PCA = 256
_HID = 256
_NCLS = 10
_CLIP = 5.0
_KBATCH = 128

_BR = 2048
_NB = _N // _BR

_SC_NC = 2
_SC_NS = 16
_SC_NW = _SC_NC * _SC_NS
_SC_ROWS = _N // _SC_NW


def _knn_block(a, ff, idx_ref, row0, d2_ref, b2_ref):
    g = lax.dot_general(a, ff, (((1,), (1,)), ((), ())),
                        preferred_element_type=jnp.float32)
    a2 = jnp.sum(a * a, axis=1, keepdims=True)
    d2_ref[...] = a2 + b2_ref[...] - 2.0 * g

    colm = lax.broadcasted_iota(jnp.int32, (_BR, _KBATCH), 1)
    rowm = lax.broadcasted_iota(jnp.int32, (_BR, _KBATCH), 0) % _KBATCH
    d2_ref[:, :_KBATCH] = jnp.where(colm == rowm, jnp.inf,
                                    d2_ref[:, :_KBATCH])

    d2 = d2_ref[...]
    mv = jnp.min(d2, axis=1, keepdims=True)
    col = lax.broadcasted_iota(jnp.int32, (_BR, _N), 1)
    cand = jnp.where(d2 == mv, col, _N)
    idx = jnp.min(cand, axis=1, keepdims=True)
    idx_ref[pl.ds(row0, _BR), :] = idx


def _norms_row(ff):
    ones_k = jnp.ones((1, ff.shape[1]), jnp.float32)
    return lax.dot_general(ones_k, ff * ff, (((1,), (1,)), ((), ())),
                           preferred_element_type=jnp.float32)


def _fused_body(xb, xf, wrf, brf, mean, comp, w0, b0, w1, b1, w2, b2, w3, b3,
                logits_ref, idx0_ref, idx1_ref, inter1_ref, d2_ref, b2_ref):
    i = pl.program_id(0)

    @pl.when(i < _NB)
    def _phase_a():
        x = xb[...]
        h = lax.dot_general(x, wrf[...], (((1,), (1,)), ((), ())),
                            preferred_element_type=jnp.float32) + brf[...]
        h = h - mean[...]
        h = lax.dot_general(h, comp[...], (((1,), (1,)), ((), ())),
                            preferred_element_type=jnp.float32)
        h = jnp.clip(h, -_CLIP, _CLIP)
        res = h
        h = jnp.maximum(
            jnp.dot(h, w0[...], preferred_element_type=jnp.float32)
            + b0[...], 0.0)
        h = jnp.dot(h, w1[...], preferred_element_type=jnp.float32) + b1[...]
        h = jnp.maximum(h + res, 0.0)
        h = jnp.maximum(
            jnp.dot(h, w2[...], preferred_element_type=jnp.float32)
            + b2[...], 0.0)
        inter1_ref[pl.ds(i * _BR, _BR), :] = h
        logits_ref[pl.ds(i * _BR, _BR), :] = (
            jnp.dot(h, w3[...], preferred_element_type=jnp.float32)
            + b3[...])

        @pl.when(i == 0)
        def _():
            b2_ref[...] = _norms_row(xf[...])

        _knn_block(x, xf[...], idx0_ref, i * _BR, d2_ref, b2_ref)

    @pl.when(i >= _NB)
    def _phase_b():
        j = i - _NB
        hh = inter1_ref[...]

        @pl.when(i == _NB)
        def _():
            b2_ref[...] = _norms_row(hh)

        a = inter1_ref[pl.ds(j * _BR, _BR), :]
        _knn_block(a, hh, idx1_ref, j * _BR, d2_ref, b2_ref)


def _run_fused(x, w_rf, b_rf, pca_mean, pca_comp, w0, b0, w1, b1, w2, b2,
               w3, b3):
    full = lambda shape: pl.BlockSpec(shape, lambda i: (0,) * len(shape))
    blk = lambda i: (i % _NB, 0)
    logits, idx0, idx1 = pl.pallas_call(
        _fused_body,
        grid=(2 * _NB,),
        in_specs=[
            pl.BlockSpec((_BR, _D_IN), blk),
            pl.BlockSpec((_N, _D_IN), lambda i: (0, 0)),
            full((_D_RF, _D_IN)),
            full((1, _D_RF)),
            full((1, _D_RF)),
            full((_D_PCA, _D_RF)),
            full((_HID, _HID)),
            full((1, _HID)),
            full((_HID, _HID)),
            full((1, _HID)),
            full((_HID, _HID)),
            full((1, _HID)),
            full((_HID, _NCLS)),
            full((1, _NCLS)),
        ],
        out_specs=[
            pl.BlockSpec((_N, _NCLS), lambda i: (0, 0)),
            pl.BlockSpec((_N, 1), lambda i: (0, 0)),
            pl.BlockSpec((_N, 1), lambda i: (0, 0)),
        ],
        out_shape=[
            jax.ShapeDtypeStruct((_N, _NCLS), jnp.float32),
            jax.ShapeDtypeStruct((_N, 1), jnp.int32),
            jax.ShapeDtypeStruct((_N, 1), jnp.int32),
        ],
        scratch_shapes=[
            pltpu.VMEM((_N, _HID), jnp.float32),
            pltpu.VMEM((_BR, _N), jnp.float32),
            pltpu.VMEM((1, _N), jnp.float32),
        ],
    )(x, x, w_rf, b_rf.reshape(1, _D_RF), pca_mean.reshape(1, _D_RF),
      pca_comp, w0, b0.reshape(1, _HID), w1, b1.reshape(1, _HID), w2,
      b2.reshape(1, _HID), w3, b3.reshape(1, _NCLS))
    return logits, idx0.reshape(_N), idx1.reshape(_N)


def _sc_bias_body(logits_hbm, idx0_hbm, idx1_hbm, y_hbm, bias_hbm, out_hbm,
                  y_v, idx_v, out_v, bias_v):
    c = lax.axis_index("c")
    s = lax.axis_index("s")
    wid = s * _SC_NC + c
    base = wid * _SC_ROWS
    pltpu.sync_copy(y_hbm, y_v)
    pltpu.sync_copy(logits_hbm.at[pl.ds(base * _NCLS, _SC_ROWS * _NCLS)],
                    out_v)
    pltpu.sync_copy(bias_hbm, bias_v)
    lane = lax.broadcasted_iota(jnp.int32, (16,), 0)
    for which, idx_hbm in enumerate((idx0_hbm, idx1_hbm)):
        pltpu.sync_copy(idx_hbm.at[pl.ds(base, _SC_ROWS)], idx_v)
        bias = bias_v[pl.ds(which * 16, 16)]
        for j in range(_SC_ROWS // 16):
            iv = idx_v[pl.ds(j * 16, 16)]
            p = plsc.load_gather(y_v, [iv])
            flat = (j * 16 + lane) * _NCLS + p
            plsc.addupdate_scatter(out_v, [flat], bias)
    pltpu.sync_copy(out_v, out_hbm.at[pl.ds(base * _NCLS, _SC_ROWS * _NCLS)])


@functools.cache
def _get_sc_bias_add():
    return pl.kernel(
        _sc_bias_body,
        mesh=plsc.VectorSubcoreMesh(core_axis_name="c",
                                    subcore_axis_name="s"),
        out_type=jax.ShapeDtypeStruct((_N * _NCLS,), jnp.float32),
        compiler_params=pltpu.CompilerParams(needs_layout_passes=False),
        scratch_types=[
            pltpu.VMEM((_N,), jnp.int32),
            pltpu.VMEM((_SC_ROWS,), jnp.int32),
            pltpu.VMEM((_SC_ROWS * _NCLS,), jnp.float32),
            pltpu.VMEM((32,), jnp.float32),
        ],
    )


def kernel(X, y, W_rf, b_rf, pca_mean, pca_comp, W0, b0, W1, b1, W2, b2,
           W3, b3, nn_bias0, nn_bias1):
    logits, idx0, idx1 = _run_fused(X, W_rf, b_rf, pca_mean, pca_comp,
                                    W0, b0, W1, b1, W2, b2, W3, b3)
    sc_bias_add = _get_sc_bias_add()
    biases = jnp.concatenate([jnp.full((16,), nn_bias0, jnp.float32),
                              jnp.full((16,), nn_bias1, jnp.float32)])
    out = sc_bias_add(logits.reshape(-1), idx0, idx1, y, biases)
    return out.reshape(_N, _NCLS)


# speedup vs baseline: 14.8572x; 1.0050x over previous
import functools

import jax
import jax.numpy as jnp
from jax import lax
from jax.experimental import pallas as pl
from jax.experimental.pallas import tpu as pltpu
from jax.experimental.pallas import tpu_sc as plsc

_N = 4096
_D_IN = 128
_D_RF = 512
_D_PCA = 256
_HID = 256
_NCLS = 10
_CLIP = 5.0
_KBATCH = 128

_BR = 2048
_NB = _N // _BR

_SC_NC = 2
_SC_NS = 16
_SC_NW = _SC_NC * _SC_NS
_SC_ROWS = _N // _SC_NW


def _knn_block(a, ff, idx_ref, row0, d2_ref, b2_ref):
    g = lax.dot_general(a, ff, (((1,), (1,)), ((), ())),
                        preferred_element_type=jnp.float32)
    a2 = jnp.sum(a * a, axis=1, keepdims=True)
    d2v = a2 + b2_ref[...] - 2.0 * g
    d2_ref[...] = d2v

    colm = lax.broadcasted_iota(jnp.int32, (_BR, _KBATCH), 1)
    rowm = lax.broadcasted_iota(jnp.int32, (_BR, _KBATCH), 0) % _KBATCH
    slab = jnp.where(colm == rowm, jnp.inf, d2v[:, :_KBATCH])
    d2_ref[:, :_KBATCH] = slab

    mv = jnp.minimum(
        jnp.min(slab, axis=1, keepdims=True),
        jnp.min(d2v[:, _KBATCH:], axis=1, keepdims=True))

    d2 = d2_ref[...]
    col = lax.broadcasted_iota(jnp.int32, (_BR, _N), 1)
    cand = jnp.where(d2 == mv, col, _N)
    idx = jnp.min(cand, axis=1, keepdims=True)
    idx_ref[pl.ds(row0, _BR), :] = idx


def _norms_row(ff):
    ones_k = jnp.ones((1, ff.shape[1]), jnp.float32)
    return lax.dot_general(ones_k, ff * ff, (((1,), (1,)), ((), ())),
                           preferred_element_type=jnp.float32)


def _fused_body(xb, xf, wrf, brf, mean, comp, w0, b0, w1, b1, w2, b2, w3, b3,
                logits_ref, idx0_ref, idx1_ref, inter1_ref, d2_ref, b2_ref):
    i = pl.program_id(0)

    @pl.when(i < _NB)
    def _phase_a():
        x = xb[...]
        h = lax.dot_general(x, wrf[...], (((1,), (1,)), ((), ())),
                            preferred_element_type=jnp.float32) + brf[...]
        h = h - mean[...]
        h = lax.dot_general(h, comp[...], (((1,), (1,)), ((), ())),
                            preferred_element_type=jnp.float32)
        h = jnp.clip(h, -_CLIP, _CLIP)
        res = h
        h = jnp.maximum(
            jnp.dot(h, w0[...], preferred_element_type=jnp.float32)
            + b0[...], 0.0)
        h = jnp.dot(h, w1[...], preferred_element_type=jnp.float32) + b1[...]
        h = jnp.maximum(h + res, 0.0)
        h = jnp.maximum(
            jnp.dot(h, w2[...], preferred_element_type=jnp.float32)
            + b2[...], 0.0)
        inter1_ref[pl.ds(i * _BR, _BR), :] = h
        logits_ref[pl.ds(i * _BR, _BR), :] = (
            jnp.dot(h, w3[...], preferred_element_type=jnp.float32)
            + b3[...])

        @pl.when(i == 0)
        def _():
            b2_ref[...] = _norms_row(xf[...])

        _knn_block(x, xf[...], idx0_ref, i * _BR, d2_ref, b2_ref)

    @pl.when(i >= _NB)
    def _phase_b():
        j = i - _NB
        hh = inter1_ref[...]

        @pl.when(i == _NB)
        def _():
            b2_ref[...] = _norms_row(hh)

        a = inter1_ref[pl.ds(j * _BR, _BR), :]
        _knn_block(a, hh, idx1_ref, j * _BR, d2_ref, b2_ref)


def _run_fused(x, w_rf, b_rf, pca_mean, pca_comp, w0, b0, w1, b1, w2, b2,
               w3, b3):
    full = lambda shape: pl.BlockSpec(shape, lambda i: (0,) * len(shape))
    blk = lambda i: (i % _NB, 0)
    logits, idx0, idx1 = pl.pallas_call(
        _fused_body,
        grid=(2 * _NB,),
        in_specs=[
            pl.BlockSpec((_BR, _D_IN), blk),
            pl.BlockSpec((_N, _D_IN), lambda i: (0, 0)),
            full((_D_RF, _D_IN)),
            full((1, _D_RF)),
            full((1, _D_RF)),
            full((_D_PCA, _D_RF)),
            full((_HID, _HID)),
            full((1, _HID)),
            full((_HID, _HID)),
            full((1, _HID)),
            full((_HID, _HID)),
            full((1, _HID)),
            full((_HID, _NCLS)),
            full((1, _NCLS)),
        ],
        out_specs=[
            pl.BlockSpec((_N, _NCLS), lambda i: (0, 0)),
            pl.BlockSpec((_N, 1), lambda i: (0, 0)),
            pl.BlockSpec((_N, 1), lambda i: (0, 0)),
        ],
        out_shape=[
            jax.ShapeDtypeStruct((_N, _NCLS), jnp.float32),
            jax.ShapeDtypeStruct((_N, 1), jnp.int32),
            jax.ShapeDtypeStruct((_N, 1), jnp.int32),
        ],
        scratch_shapes=[
            pltpu.VMEM((_N, _HID), jnp.float32),
            pltpu.VMEM((_BR, _N), jnp.float32),
            pltpu.VMEM((1, _N), jnp.float32),
        ],
    )(x, x, w_rf, b_rf.reshape(1, _D_RF), pca_mean.reshape(1, _D_RF),
      pca_comp, w0, b0.reshape(1, _HID), w1, b1.reshape(1, _HID), w2,
      b2.reshape(1, _HID), w3, b3.reshape(1, _NCLS))
    return logits, idx0.reshape(_N), idx1.reshape(_N)


def _sc_bias_body(logits_hbm, idx0_hbm, idx1_hbm, y_hbm, bias_hbm, out_hbm,
                  y_v, idx_v, out_v, bias_v):
    c = lax.axis_index("c")
    s = lax.axis_index("s")
    wid = s * _SC_NC + c
    base = wid * _SC_ROWS
    pltpu.sync_copy(y_hbm, y_v)
    pltpu.sync_copy(logits_hbm.at[pl.ds(base * _NCLS, _SC_ROWS * _NCLS)],
                    out_v)
    pltpu.sync_copy(bias_hbm, bias_v)
    lane = lax.broadcasted_iota(jnp.int32, (16,), 0)
    for which, idx_hbm in enumerate((idx0_hbm, idx1_hbm)):
        pltpu.sync_copy(idx_hbm.at[pl.ds(base, _SC_ROWS)], idx_v)
        bias = bias_v[pl.ds(which * 16, 16)]
        for j in range(_SC_ROWS // 16):
            iv = idx_v[pl.ds(j * 16, 16)]
            p = plsc.load_gather(y_v, [iv])
            flat = (j * 16 + lane) * _NCLS + p
            plsc.addupdate_scatter(out_v, [flat], bias)
    pltpu.sync_copy(out_v, out_hbm.at[pl.ds(base * _NCLS, _SC_ROWS * _NCLS)])


@functools.cache
def _get_sc_bias_add():
    return pl.kernel(
        _sc_bias_body,
        mesh=plsc.VectorSubcoreMesh(core_axis_name="c",
                                    subcore_axis_name="s"),
        out_type=jax.ShapeDtypeStruct((_N * _NCLS,), jnp.float32),
        compiler_params=pltpu.CompilerParams(needs_layout_passes=False),
        scratch_types=[
            pltpu.VMEM((_N,), jnp.int32),
            pltpu.VMEM((_SC_ROWS,), jnp.int32),
            pltpu.VMEM((_SC_ROWS * _NCLS,), jnp.float32),
            pltpu.VMEM((32,), jnp.float32),
        ],
    )


def kernel(X, y, W_rf, b_rf, pca_mean, pca_comp, W0, b0, W1, b1, W2, b2,
           W3, b3, nn_bias0, nn_bias1):
    logits, idx0, idx1 = _run_fused(X, W_rf, b_rf, pca_mean, pca_comp,
                                    W0, b0, W1, b1, W2, b2, W3, b3)
    sc_bias_add = _get_sc_bias_add()
    biases = jnp.concatenate([jnp.full((16,), nn_bias0, jnp.float32),
                              jnp.full((16,), nn_bias1, jnp.float32)])
    out = sc_bias_add(logits.reshape(-1), idx0, idx1, y, biases)
    return out.reshape(_N, _NCLS)
